# Initial kernel scaffold; baseline (speedup 1.0000x reference)
import jax
import jax.numpy as jnp
from jax import lax
from jax.experimental import pallas as pl
from jax.experimental.pallas import tpu as pltpu
from jax.experimental.pallas import tpu_sc as plsc

_B, _H, _D = 4096, 200, 32
_N = _B * _H
_NC, _NS = 2, 16
_NW = _NC * _NS
_NPW = _N // _NW
_CHUNK = 128
_NCH = _NPW // _CHUNK


def _emb_body(idx_hbm, table_hbm, out_hbm, idx_v, row_v, sem):
    wid = lax.axis_index("s") * _NC + lax.axis_index("c")
    base = wid * _NPW
    pltpu.sync_copy(idx_hbm.at[wid], idx_v)

    def chunk(j, carry):
        pltpu.async_copy(table_hbm.at[idx_v.at[j]], row_v, sem).wait()
        pltpu.sync_copy(row_v, out_hbm.at[pl.ds(base + j * _CHUNK, _CHUNK)])
        return carry

    lax.fori_loop(0, _NCH, chunk, 0)


@jax.jit
def kernel(batch_seq, table):
    idx = batch_seq.astype(jnp.int32).reshape(_NW, _NCH, _CHUNK)
    k = pl.kernel(
        _emb_body,
        out_type=jax.ShapeDtypeStruct((_N, _D), jnp.float32),
        mesh=plsc.VectorSubcoreMesh(core_axis_name="c", subcore_axis_name="s"),
        scratch_types=[
            pltpu.VMEM((_NCH, _CHUNK), jnp.int32),
            pltpu.VMEM((_CHUNK, _D), jnp.float32),
            pltpu.SemaphoreType.DMA,
        ],
    )
    out = k(idx, table)
    return out.reshape(_B, _H, _D)


# speedup vs baseline: 1.3063x; 1.3063x over previous
import jax
import jax.numpy as jnp
from jax import lax
from jax.experimental import pallas as pl
from jax.experimental.pallas import tpu as pltpu
from jax.experimental.pallas import tpu_sc as plsc

_B, _H, _D = 4096, 200, 32
_N = _B * _H
_NC, _NS = 2, 16
_NW = _NC * _NS
_NPW = _N // _NW
_CHUNK = 128
_NCH = _NPW // _CHUNK


def _emb_body(idx_hbm, table_hbm, out_hbm, idx_v, row_v, sem):
    wid = lax.axis_index("s") * _NC + lax.axis_index("c")
    base = wid * _NPW
    pltpu.sync_copy(idx_hbm.at[wid], idx_v)

    def chunk(j, carry):
        pltpu.async_copy(table_hbm.at[idx_v.at[j]], row_v, sem).wait()
        pltpu.sync_copy(row_v, out_hbm.at[pl.ds(base + j * _CHUNK, _CHUNK)])
        return carry

    lax.fori_loop(0, _NCH, chunk, 0)


@jax.jit
def kernel(batch_seq, table):
    idx = batch_seq.astype(jnp.int32).reshape(_NW, _NCH, _CHUNK)
    k = pl.kernel(
        _emb_body,
        out_type=jax.ShapeDtypeStruct((_N, _D), jnp.float32),
        mesh=plsc.VectorSubcoreMesh(core_axis_name="c", subcore_axis_name="s"),
        scratch_types=[
            pltpu.VMEM((_NCH, _CHUNK), jnp.int32),
            pltpu.VMEM((_CHUNK, _D), jnp.float32),
            pltpu.SemaphoreType.DMA,
        ],
        compiler_params=pltpu.CompilerParams(use_tc_tiling_on_sc=False),
    )
    out = k(idx, table)
    return out.reshape(_B, _H, _D)


# speedup vs baseline: 1.5004x; 1.1486x over previous
import jax
import jax.numpy as jnp
from jax import lax
from jax.experimental import pallas as pl
from jax.experimental.pallas import tpu as pltpu
from jax.experimental.pallas import tpu_sc as plsc

_B, _H, _D = 4096, 200, 32
_N = _B * _H
_NC, _NS = 2, 16
_NW = _NC * _NS
_NPW = _N // _NW
_CHUNK = 128
_NCH = _NPW // _CHUNK
_NBUF = 8
_LA = 4


def _emb_body(idx_hbm, table_hbm, out_hbm, idx_v, buf, gsem, wsem):
    wid = lax.axis_index("s") * _NC + lax.axis_index("c")
    base = wid * _NPW
    pltpu.sync_copy(idx_hbm.at[wid], idx_v)

    def gather_start(j, s):
        pltpu.make_async_copy(
            table_hbm.at[idx_v.at[j]], buf.at[s], gsem.at[s]).start()

    def gather_wait(s):
        pltpu.make_async_copy(
            table_hbm.at[idx_v.at[0]], buf.at[s], gsem.at[s]).wait()

    def write_start(j, s):
        pltpu.make_async_copy(
            buf.at[s], out_hbm.at[pl.ds(base + j * _CHUNK, _CHUNK)],
            wsem.at[s]).start()

    def write_wait(s):
        pltpu.make_async_copy(
            buf.at[s], out_hbm.at[pl.ds(base, _CHUNK)], wsem.at[s]).wait()

    def step(j, b, first_round, last_round):
        sf = (b + _LA) % _NBUF
        if not last_round:
            if not (first_round and b < _NBUF - _LA):
                write_wait(sf)
            gather_start(j + _LA, sf)
        elif b < _NBUF - _LA:
            write_wait(sf)
            gather_start(j + _LA, sf)
        gather_wait(b)
        write_start(j, b)

    for j in range(_LA):
        gather_start(j, j)

    for b in range(_NBUF):
        step(b, b, True, False)

    @pl.loop(_NBUF, _NCH - _NBUF, step=_NBUF)
    def _round(j0):
        for b in range(_NBUF):
            step(j0 + b, b, False, False)

    for b in range(_NBUF):
        step(_NCH - _NBUF + b, b, False, True)

    for s in range(_NBUF):
        write_wait(s)


@jax.jit
def kernel(batch_seq, table):
    idx = batch_seq.astype(jnp.int32).reshape(_NW, _NCH, _CHUNK)
    k = pl.kernel(
        _emb_body,
        out_type=jax.ShapeDtypeStruct((_N, _D), jnp.float32),
        mesh=plsc.VectorSubcoreMesh(core_axis_name="c", subcore_axis_name="s"),
        scratch_types=[
            pltpu.VMEM((_NCH, _CHUNK), jnp.int32),
            pltpu.VMEM((_NBUF, _CHUNK, _D), jnp.float32),
            pltpu.SemaphoreType.DMA((_NBUF,)),
            pltpu.SemaphoreType.DMA((_NBUF,)),
        ],
        compiler_params=pltpu.CompilerParams(use_tc_tiling_on_sc=False),
    )
    out = k(idx, table)
    return out.reshape(_B, _H, _D)
